# baseline (speedup 1.0000x reference)
import functools

import jax
import jax.numpy as jnp
import numpy as np
from jax import lax
from jax.experimental import pallas as pl
from jax.experimental.pallas import tpu as pltpu
from jax.experimental.pallas import tpu_sc as plsc

_TOP_P = 0.95
_BRACKET = 17.0
_NB = 1024
_V = 100000
_NVREG = _V // 16
_UNROLL = 10
_ROWS_PER_W = 4

_G = np.asarray(jax.random.gumbel(jax.random.key(42), (128, _V), jnp.float32))


def _lane():
    return lax.iota(jnp.int32, 16)


_GDN = lax.GatherDimensionNumbers(
    offset_dims=(), collapsed_slice_dims=(0,), start_index_map=(0,))


def _permute(v, idx):
    return lax.gather(v, idx[:, None], _GDN, slice_sizes=(1,),
                      mode=lax.GatherScatterMode.PROMISE_IN_BOUNDS)


def _shuf(v, sh):
    return _permute(v, (_lane() + sh) & 15)


def _bmax(v):
    for sh in (1, 2, 4, 8):
        v = jnp.maximum(v, _shuf(v, sh))
    return v


def _badd(v):
    for sh in (1, 2, 4, 8):
        v = v + _shuf(v, sh)
    return v


def _last_splat(v):
    return _permute(v, jnp.full((16,), 15, jnp.int32))


def _merge(hist, merged):
    def body(j, sacc):
        t = hist[pl.ds(j * 16, 16)]
        for l in range(1, 16):
            t = t + hist[pl.ds(l * _NB + j * 16, 16)]
        merged[pl.ds(j * 16, 16)] = t
        return sacc + t

    sacc = lax.fori_loop(0, _NB // 16, body, jnp.zeros((16,), jnp.float32))
    return _badd(sacc)


def _scan(merged, s_above, target):
    def body(j, carry):
        base, cnt, acap = carry
        t = merged[pl.ds(j * 16, 16)]
        c = plsc.cumsum(t)
        above = s_above - (base + c)
        mask = above > target
        cnt = cnt + plsc.all_reduce_population_count(mask)
        asel = jnp.where(mask, jnp.float32(-1.0), above)
        acap = jnp.maximum(acap, asel)
        return base + _last_splat(c), cnt, acap

    _, cnt, acap = lax.fori_loop(
        0, _NB // 16, body,
        (jnp.zeros((16,), jnp.float32), jnp.zeros((16,), jnp.int32),
         jnp.full((16,), -1.0, jnp.float32)))
    return cnt, _bmax(acap)


def _sc_body(x_hbm, t_hbm, xv, hist, merged, tstage):
    wid = lax.axis_index("s") * 2 + lax.axis_index("c")
    lane = _lane()
    lbase = lane * _NB
    zeros16 = jnp.zeros((16,), jnp.float32)
    w1 = np.float32(_BRACKET / _NB)
    w2 = np.float32(_BRACKET / _NB / _NB)

    tstage[...] = zeros16

    def zero_hist(j, _):
        for l in range(16):
            hist[pl.ds(l * _NB + j * 16, 16)] = zeros16
        return 0

    for k in range(_ROWS_PER_W):
        row = wid * _ROWS_PER_W + k
        pltpu.sync_copy(x_hbm.at[row], xv)

        def mx_body(j, accs):
            return tuple(
                jnp.maximum(accs[u], xv[pl.ds((j * _UNROLL + u) * 16, 16)])
                for u in range(_UNROLL))

        accs = lax.fori_loop(
            0, _NVREG // _UNROLL, mx_body,
            tuple(jnp.full((16,), -jnp.inf, jnp.float32) for _ in range(_UNROLL)))
        m16 = accs[0]
        for u in range(1, _UNROLL):
            m16 = jnp.maximum(m16, accs[u])
        m = _bmax(m16)

        lo1 = m - np.float32(_BRACKET)
        s1 = np.float32(_NB / _BRACKET)
        lax.fori_loop(0, _NB // 16, zero_hist, 0)

        def h1_body(j, _):
            for u in range(_UNROLL):
                xc = xv[pl.ds((j * _UNROLL + u) * 16, 16)]
                e = jnp.exp(xc - m)
                b = jnp.clip(((xc - lo1) * s1).astype(jnp.int32), 0, _NB - 1)
                plsc.addupdate_scatter(hist, [b * 16 + lane], e)
            return 0

        lax.fori_loop(0, _NVREG // _UNROLL, h1_body, 0)
        s_full = _merge(hist, merged)
        target = jnp.float32(_TOP_P) * s_full
        bstar, a1 = _scan(merged, s_full, target)

        lo2 = lo1 + bstar.astype(jnp.float32) * w1
        hi2 = lo2 + w1
        s2 = np.float32(_NB) / w1

        lax.fori_loop(0, _NB // 16, zero_hist, 0)

        def h2_body(j, _):
            for u in range(_UNROLL):
                xc = xv[pl.ds((j * _UNROLL + u) * 16, 16)]
                e = jnp.exp(xc - m)
                inb = (xc >= lo2) & (xc < hi2)
                b = jnp.clip(((xc - lo2) * s2).astype(jnp.int32), 0, _NB - 1)
                plsc.addupdate_scatter(hist, [b * 16 + lane], e, mask=inb)
            return 0

        lax.fori_loop(0, _NVREG // _UNROLL, h2_body, 0)
        s_bucket = _merge(hist, merged)
        cstar, _ = _scan(merged, a1 + s_bucket, target)

        t_hat = lo2 + cstar.astype(jnp.float32) * w2
        tstage[...] = jnp.where(lane == k, t_hat, tstage[...])

    pltpu.sync_copy(tstage, t_hbm.at[wid])


def _sc_thresholds(logits):
    mesh = plsc.VectorSubcoreMesh(core_axis_name="c", subcore_axis_name="s")
    f = functools.partial(
        pl.kernel,
        mesh=mesh,
        compiler_params=pltpu.CompilerParams(needs_layout_passes=False),
        out_type=jax.ShapeDtypeStruct((32, 16), jnp.float32),
        scratch_types=[
            pltpu.VMEM((_V,), jnp.float32),
            pltpu.VMEM((16 * _NB,), jnp.float32),
            pltpu.VMEM((_NB,), jnp.float32),
            pltpu.VMEM((16,), jnp.float32),
        ],
    )(_sc_body)
    return f(logits)


def _bounds(v, n=8):
    w = (-(-v // n) + 127) // 128 * 128
    return [(a, min(a + w, v)) for a in range(0, v, w)]


def _rmax(a):
    parts = [jnp.max(a[:, s:t], axis=1, keepdims=True) for s, t in _bounds(a.shape[1])]
    out = parts[0]
    for p in parts[1:]:
        out = jnp.maximum(out, p)
    return out


def _rmin(a):
    parts = [jnp.min(a[:, s:t], axis=1, keepdims=True) for s, t in _bounds(a.shape[1])]
    out = parts[0]
    for p in parts[1:]:
        out = jnp.minimum(out, p)
    return out


def _am_body(x_ref, g_ref, t_ref, o_ref):
    x = x_ref[...]
    v = x.shape[1]
    y = jnp.where(x >= t_ref[...], x + g_ref[...], -jnp.inf)
    best = _rmax(y)
    ids = lax.broadcasted_iota(jnp.int32, x.shape, 1)
    idx = _rmin(jnp.where(y == best, ids, jnp.int32(v)))
    o_ref[...] = idx.astype(jnp.int32)


def _argmax_kernel(logits, gumbel, t_rows):
    b, v = logits.shape
    r = 8
    return pl.pallas_call(
        _am_body,
        grid=(b // r,),
        in_specs=[
            pl.BlockSpec((r, v), lambda i: (i, 0)),
            pl.BlockSpec((r, v), lambda i: (i, 0)),
            pl.BlockSpec((r, 1), lambda i: (i, 0)),
        ],
        out_specs=pl.BlockSpec((r, 1), lambda i: (i, 0)),
        out_shape=jax.ShapeDtypeStruct((b, 1), jnp.int32),
    )(logits, gumbel, t_rows)


def kernel(logits):
    t = _sc_thresholds(logits)
    t_rows = t[:, :_ROWS_PER_W].reshape(-1, 1)
    return _argmax_kernel(logits, _G, t_rows)


# speedup vs baseline: 2.7632x; 2.7632x over previous
import functools

import jax
import jax.numpy as jnp
import numpy as np
from jax import lax
from jax.experimental import pallas as pl
from jax.experimental.pallas import tpu as pltpu
from jax.experimental.pallas import tpu_sc as plsc

_TOP_P = 0.95
_BRACKET = 17.0
_NB = 1024
_V = 100000
_NVREG = _V // 16
_UNROLL = 10
_ROWS_PER_W = 4

_G = np.asarray(jax.random.gumbel(jax.random.key(42), (128, _V), jnp.float32))


def _lane():
    return lax.iota(jnp.int32, 16)


_GDN = lax.GatherDimensionNumbers(
    offset_dims=(), collapsed_slice_dims=(0,), start_index_map=(0,))


def _permute(v, idx):
    return lax.gather(v, idx[:, None], _GDN, slice_sizes=(1,),
                      mode=lax.GatherScatterMode.PROMISE_IN_BOUNDS)


def _shuf(v, sh):
    return _permute(v, (_lane() + sh) & 15)


def _bmax(v):
    for sh in (1, 2, 4, 8):
        v = jnp.maximum(v, _shuf(v, sh))
    return v


def _badd(v):
    for sh in (1, 2, 4, 8):
        v = v + _shuf(v, sh)
    return v


def _last_splat(v):
    return _permute(v, jnp.full((16,), 15, jnp.int32))


def _merge(hist, merged):
    def body(j, sacc):
        t = hist[pl.ds(j * 16, 16)]
        for l in range(1, 16):
            t = t + hist[pl.ds(l * _NB + j * 16, 16)]
        merged[pl.ds(j * 16, 16)] = t
        return sacc + t

    sacc = lax.fori_loop(0, _NB // 16, body, jnp.zeros((16,), jnp.float32))
    return _badd(sacc)


def _scan(merged, s_above, target):
    def body(j, carry):
        base, cnt, acap = carry
        t = merged[pl.ds(j * 16, 16)]
        c = plsc.cumsum(t)
        above = s_above - (base + c)
        mask = above > target
        cnt = cnt + plsc.all_reduce_population_count(mask)
        asel = jnp.where(mask, jnp.float32(-1.0), above)
        acap = jnp.maximum(acap, asel)
        return base + _last_splat(c), cnt, acap

    _, cnt, acap = lax.fori_loop(
        0, _NB // 16, body,
        (jnp.zeros((16,), jnp.float32), jnp.zeros((16,), jnp.int32),
         jnp.full((16,), -1.0, jnp.float32)))
    return cnt, _bmax(acap)


def _sc_body(x_hbm, t_hbm, xv, hist, merged, tstage):
    wid = lax.axis_index("s") * 2 + lax.axis_index("c")
    lane = _lane()
    lbase = lane * _NB
    zeros16 = jnp.zeros((16,), jnp.float32)
    w1 = np.float32(_BRACKET / _NB)
    w2 = np.float32(_BRACKET / _NB / _NB)

    tstage[...] = zeros16

    def zero_hist(j, _):
        for l in range(16):
            hist[pl.ds(l * _NB + j * 16, 16)] = zeros16
        return 0

    for k in range(_ROWS_PER_W):
        row = wid * _ROWS_PER_W + k
        pltpu.sync_copy(x_hbm.at[row], xv)

        def mx_body(j, accs):
            return tuple(
                jnp.maximum(accs[u], xv[pl.ds((j * _UNROLL + u) * 16, 16)])
                for u in range(_UNROLL))

        accs = lax.fori_loop(
            0, _NVREG // _UNROLL, mx_body,
            tuple(jnp.full((16,), -jnp.inf, jnp.float32) for _ in range(_UNROLL)))
        m16 = accs[0]
        for u in range(1, _UNROLL):
            m16 = jnp.maximum(m16, accs[u])
        m = _bmax(m16)

        lo1 = m - np.float32(_BRACKET)
        s1 = np.float32(_NB / _BRACKET)
        lax.fori_loop(0, _NB // 16, zero_hist, 0)

        @plsc.parallel_loop(0, _NVREG, unroll=_UNROLL)
        def h1_body(j):
            xc = xv[pl.ds(j * 16, 16)]
            e = jnp.exp(xc - m)
            b = jnp.clip(((xc - lo1) * s1).astype(jnp.int32), 0, _NB - 1)
            plsc.addupdate_scatter(hist, [lbase + b], e)
        s_full = _merge(hist, merged)
        target = jnp.float32(_TOP_P) * s_full
        bstar, a1 = _scan(merged, s_full, target)

        lo2 = lo1 + bstar.astype(jnp.float32) * w1
        hi2 = lo2 + w1
        s2 = np.float32(_NB) / w1

        lax.fori_loop(0, _NB // 16, zero_hist, 0)

        @plsc.parallel_loop(0, _NVREG, unroll=_UNROLL)
        def h2_body(j):
            xc = xv[pl.ds(j * 16, 16)]
            e = jnp.exp(xc - m)
            inb = (xc >= lo2) & (xc < hi2)
            b = jnp.clip(((xc - lo2) * s2).astype(jnp.int32), 0, _NB - 1)
            plsc.addupdate_scatter(hist, [lbase + b], e, mask=inb)
        s_bucket = _merge(hist, merged)
        cstar, _ = _scan(merged, a1 + s_bucket, target)

        t_hat = lo2 + cstar.astype(jnp.float32) * w2
        tstage[...] = jnp.where(lane == k, t_hat, tstage[...])

    pltpu.sync_copy(tstage, t_hbm.at[wid])


def _sc_thresholds(logits):
    mesh = plsc.VectorSubcoreMesh(core_axis_name="c", subcore_axis_name="s")
    f = functools.partial(
        pl.kernel,
        mesh=mesh,
        compiler_params=pltpu.CompilerParams(needs_layout_passes=False),
        out_type=jax.ShapeDtypeStruct((32, 16), jnp.float32),
        scratch_types=[
            pltpu.VMEM((_V,), jnp.float32),
            pltpu.VMEM((16 * _NB,), jnp.float32),
            pltpu.VMEM((_NB,), jnp.float32),
            pltpu.VMEM((16,), jnp.float32),
        ],
    )(_sc_body)
    return f(logits)


def _bounds(v, n=8):
    w = (-(-v // n) + 127) // 128 * 128
    return [(a, min(a + w, v)) for a in range(0, v, w)]


def _rmax(a):
    parts = [jnp.max(a[:, s:t], axis=1, keepdims=True) for s, t in _bounds(a.shape[1])]
    out = parts[0]
    for p in parts[1:]:
        out = jnp.maximum(out, p)
    return out


def _rmin(a):
    parts = [jnp.min(a[:, s:t], axis=1, keepdims=True) for s, t in _bounds(a.shape[1])]
    out = parts[0]
    for p in parts[1:]:
        out = jnp.minimum(out, p)
    return out


def _am_body(x_ref, g_ref, t_ref, o_ref):
    x = x_ref[...]
    v = x.shape[1]
    y = jnp.where(x >= t_ref[...], x + g_ref[...], -jnp.inf)
    best = _rmax(y)
    ids = lax.broadcasted_iota(jnp.int32, x.shape, 1)
    idx = _rmin(jnp.where(y == best, ids, jnp.int32(v)))
    o_ref[...] = idx.astype(jnp.int32)


def _argmax_kernel(logits, gumbel, t_rows):
    b, v = logits.shape
    r = 8
    return pl.pallas_call(
        _am_body,
        grid=(b // r,),
        in_specs=[
            pl.BlockSpec((r, v), lambda i: (i, 0)),
            pl.BlockSpec((r, v), lambda i: (i, 0)),
            pl.BlockSpec((r, 1), lambda i: (i, 0)),
        ],
        out_specs=pl.BlockSpec((r, 1), lambda i: (i, 0)),
        out_shape=jax.ShapeDtypeStruct((b, 1), jnp.int32),
    )(logits, gumbel, t_rows)


def kernel(logits):
    t = _sc_thresholds(logits)
    t_rows = t[:, :_ROWS_PER_W].reshape(-1, 1)
    return _argmax_kernel(logits, _G, t_rows)
